# Initial kernel scaffold; baseline (speedup 1.0000x reference)
import jax
import jax.numpy as jnp
from jax.experimental import pallas as pl


def kernel(input):
    raise NotImplementedError("write your pallas kernel here")


# speedup vs baseline: 653.5834x; 653.5834x over previous
import functools
import math

import jax
import jax.numpy as jnp
import numpy as np
from jax import lax
from jax.experimental import pallas as pl
from jax.experimental.pallas import tpu as pltpu
from jax.experimental.pallas import tpu_sc as plsc

B, C, H, W = 16, 3, 512, 512
GH = GW = 8
TS = 64
HALF = TS // 2
NBINS = 256
PIXELS = TS * TS
MAXV = 640.0
LUT_SCALE = (NBINS - 1) / PIXELS

NIMG = B * C
IMG_PIX = H * W
TROW_PIX = TS * W
NTROW = NIMG * GH
TROW_PER_TEC = NTROW // 32
LUT_PER_TROW = GW * NBINS
LUT_TOTAL = NTROW * LUT_PER_TROW

K1_CHUNK = TROW_PIX // 2
NHALF = NIMG * 2
HALF_PER_TEC = NHALF // 32
HALF_PIX = IMG_PIX // 2
K2_CHUNK = 32 * W
K2_NCHUNK = HALF_PIX // K2_CHUNK
LUT_BLK = 5 * LUT_PER_TROW

_MESH = plsc.VectorSubcoreMesh(
    core_axis_name="c", subcore_axis_name="s", num_cores=2, num_subcores=16)


def _axis_tables(n_pix, half, n_tiles):
    pos = np.arange(n_pix)
    m = pos // half
    last = 2 * n_tiles - 1
    interior = (m > 0) & (m < last)
    p = np.clip((m - 1) // 2, 0, n_tiles - 2)
    i0 = np.where(m == 0, 0, np.where(m == last, n_tiles - 1, p))
    i1 = np.where(m == 0, 0, np.where(m == last, n_tiles - 1,
                                      np.minimum(p + 1, n_tiles - 1)))
    r = (pos - (2 * p + 1) * half).astype(np.float32)
    denom = np.float32(2 * half - 1)
    w = np.where(interior, (denom - r) / denom, np.float32(1.0)).astype(np.float32)
    return (i0.astype(np.int32) * NBINS, i1.astype(np.int32) * NBINS,
            w, (np.float32(1.0) - w))


def _hist_lut_body(x_hbm, lut_hbm, inb, hist, lutb):
    cid = lax.axis_index("c")
    sid = lax.axis_index("s")
    wid = sid * 2 + cid
    ones = jnp.full((16,), 1.0, jnp.float32)
    iota_f = lax.iota(jnp.int32, 16).astype(jnp.float32)

    def per_tilerow(t, _):
        tr = wid * TROW_PER_TEC + t
        base = tr * TROW_PIX

        def zero(k, _c):
            hist[pl.ds(k * 16, 16)] = jnp.zeros((16,), jnp.float32)
            return 0
        lax.fori_loop(0, LUT_PER_TROW // 16, zero, 0)

        def per_chunk(ch, _c):
            pltpu.sync_copy(x_hbm.at[pl.ds(base + ch * K1_CHUNK, K1_CHUNK)], inb)

            def per_vreg(g, _c2):
                def one(gg):
                    x = inb[pl.ds(gg * 16, 16)]
                    bins = jnp.clip((x * 256.0).astype(jnp.int32), 0, 255)
                    i = (gg % 32) // 4
                    plsc.addupdate_scatter(hist, [bins + i * NBINS], ones)
                one(2 * g)
                one(2 * g + 1)
                return 0
            lax.fori_loop(0, K1_CHUNK // 32, per_vreg, 0)
            return 0
        lax.fori_loop(0, TROW_PIX // K1_CHUNK, per_chunk, 0)

        def per_tile(i, _c):
            hbase = i * NBINS

            def clip_sum(k, acc):
                h = jnp.minimum(hist[pl.ds(hbase + k * 16, 16)], MAXV)
                hist[pl.ds(hbase + k * 16, 16)] = h
                return acc + h
            accv = lax.fori_loop(0, 16, clip_sum,
                                 jnp.zeros((16,), jnp.float32))
            clipped = float(PIXELS) - jnp.sum(accv)
            q = (clipped * (1.0 / NBINS)).astype(jnp.int32).astype(jnp.float32)
            residual = clipped - q * float(NBINS)

            def cdf(k, carry):
                h = hist[pl.ds(hbase + k * 16, 16)]
                ind = jnp.where(iota_f + k.astype(jnp.float32) * 16.0 < residual,
                                1.0, 0.0)
                h2 = h + q + ind
                cs = plsc.cumsum(h2) + carry
                lv = jnp.clip(cs * LUT_SCALE, 0.0, 255.0)
                lutb[pl.ds(hbase + k * 16, 16)] = (
                    lv.astype(jnp.int32).astype(jnp.float32))
                return carry + jnp.sum(h2)
            lax.fori_loop(0, 16, cdf, jnp.float32(0.0))
            return 0
        lax.fori_loop(0, GW, per_tile, 0)

        pltpu.sync_copy(lutb, lut_hbm.at[pl.ds(tr * LUT_PER_TROW, LUT_PER_TROW)])
        return 0
    lax.fori_loop(0, TROW_PER_TEC, per_tilerow, 0)


def _interp_body(x_hbm, lut_hbm, ci0_h, ci1_h, wx_h, wx1_h, out_hbm,
                 inb, outb, lutv, ci0, ci1, wxv, wx1v):
    cid = lax.axis_index("c")
    sid = lax.axis_index("s")
    wid = sid * 2 + cid
    pltpu.sync_copy(ci0_h, ci0)
    pltpu.sync_copy(ci1_h, ci1)
    pltpu.sync_copy(wx_h, wxv)
    pltpu.sync_copy(wx1_h, wx1v)

    def per_half(hq, _):
        hw = wid * HALF_PER_TEC + hq
        img = hw // 2
        half = hw % 2
        jbase = half * 3
        pltpu.sync_copy(
            lut_hbm.at[pl.ds(img * GH * LUT_PER_TROW + jbase * LUT_PER_TROW,
                             LUT_BLK)], lutv)
        pixbase = img * IMG_PIX + half * HALF_PIX

        def per_chunk(ch, _c):
            off = pixbase + ch * K2_CHUNK
            pltpu.sync_copy(x_hbm.at[pl.ds(off, K2_CHUNK)], inb)

            def per_row(rl, _c2):
                r = half * 256 + ch * 32 + rl
                m = r // HALF
                interior = jnp.logical_and(m > 0, m < 15)
                p = jnp.clip((m - 1) // 2, 0, GH - 2)
                j0 = jnp.where(m == 0, 0, jnp.where(m == 15, GH - 1, p))
                j1 = jnp.where(m == 0, 0,
                               jnp.where(m == 15, GH - 1,
                                         jnp.minimum(p + 1, GH - 1)))
                rr = (r - (2 * p + 1) * HALF).astype(jnp.float32)
                wy = jnp.where(interior, (63.0 - rr) * (1.0 / 63.0), 1.0)
                wy1 = 1.0 - wy
                ro0 = (j0 - jbase) * LUT_PER_TROW
                ro1 = (j1 - jbase) * LUT_PER_TROW
                rowb = rl * W

                def per_g(g, _c3):
                    x = inb[pl.ds(rowb + g * 16, 16)]
                    v = (x * 255.0).astype(jnp.int32)
                    u0 = v + ci0[pl.ds(g * 16, 16)]
                    u1 = v + ci1[pl.ds(g * 16, 16)]
                    o00 = plsc.load_gather(lutv, [u0 + ro0])
                    o01 = plsc.load_gather(lutv, [u1 + ro0])
                    o10 = plsc.load_gather(lutv, [u0 + ro1])
                    o11 = plsc.load_gather(lutv, [u1 + ro1])
                    wx = wxv[pl.ds(g * 16, 16)]
                    wx1 = wx1v[pl.ds(g * 16, 16)]
                    m0 = wx * o00 + wx1 * o01
                    m1 = wx * o10 + wx1 * o11
                    outb[pl.ds(rowb + g * 16, 16)] = (
                        (wy * m0 + wy1 * m1) * (1.0 / 255.0))
                    return 0
                lax.fori_loop(0, W // 16, per_g, 0)
                return 0
            lax.fori_loop(0, K2_CHUNK // W, per_row, 0)
            pltpu.sync_copy(outb, out_hbm.at[pl.ds(off, K2_CHUNK)])
            return 0
        lax.fori_loop(0, K2_NCHUNK, per_chunk, 0)
        return 0
    lax.fori_loop(0, HALF_PER_TEC, per_half, 0)


_SC_PARAMS = pltpu.CompilerParams(needs_layout_passes=False)

_hist_lut = pl.kernel(
    _hist_lut_body,
    out_type=jax.ShapeDtypeStruct((LUT_TOTAL,), jnp.float32),
    mesh=_MESH,
    compiler_params=_SC_PARAMS,
    scratch_types=[
        pltpu.VMEM((K1_CHUNK,), jnp.float32),
        pltpu.VMEM((LUT_PER_TROW,), jnp.float32),
        pltpu.VMEM((LUT_PER_TROW,), jnp.float32),
    ],
)

_interp = pl.kernel(
    _interp_body,
    out_type=jax.ShapeDtypeStruct((NIMG * IMG_PIX,), jnp.float32),
    mesh=_MESH,
    compiler_params=_SC_PARAMS,
    scratch_types=[
        pltpu.VMEM((K2_CHUNK,), jnp.float32),
        pltpu.VMEM((K2_CHUNK,), jnp.float32),
        pltpu.VMEM((LUT_BLK,), jnp.float32),
        pltpu.VMEM((W,), jnp.int32),
        pltpu.VMEM((W,), jnp.int32),
        pltpu.VMEM((W,), jnp.float32),
        pltpu.VMEM((W,), jnp.float32),
    ],
)


def kernel(input):
    x_flat = input.reshape(-1)
    ci0, ci1, wx, wx1 = _axis_tables(W, HALF, GW)
    luts = _hist_lut(x_flat)
    out = _interp(x_flat, luts, jnp.asarray(ci0), jnp.asarray(ci1),
                  jnp.asarray(wx), jnp.asarray(wx1))
    return out.reshape(input.shape)


# speedup vs baseline: 1242.5622x; 1.9012x over previous
import functools
import math

import jax
import jax.numpy as jnp
import numpy as np
from jax import lax
from jax.experimental import pallas as pl
from jax.experimental.pallas import tpu as pltpu
from jax.experimental.pallas import tpu_sc as plsc

B, C, H, W = 16, 3, 512, 512
GH = GW = 8
TS = 64
HALF = TS // 2
NBINS = 256
PIXELS = TS * TS
MAXV = 640.0
LUT_SCALE = (NBINS - 1) / PIXELS

NIMG = B * C
IMG_PIX = H * W
TROW_PIX = TS * W
NTROW = NIMG * GH
TROW_PER_TEC = NTROW // 32
LUT_PER_TROW = GW * NBINS
LUT_TOTAL = NTROW * LUT_PER_TROW

K1_CHUNK = TROW_PIX // 2
NHALF = NIMG * 2
HALF_PER_TEC = NHALF // 32
HALF_PIX = IMG_PIX // 2
K2_CHUNK = 32 * W
K2_NCHUNK = HALF_PIX // K2_CHUNK
LUT_BLK = 5 * LUT_PER_TROW

_MESH = plsc.VectorSubcoreMesh(
    core_axis_name="c", subcore_axis_name="s", num_cores=2, num_subcores=16)
_SC_PARAMS = pltpu.CompilerParams(needs_layout_passes=False)


def _axis_tables(n_pix, half, n_tiles):
    pos = np.arange(n_pix)
    m = pos // half
    last = 2 * n_tiles - 1
    interior = (m > 0) & (m < last)
    p = np.clip((m - 1) // 2, 0, n_tiles - 2)
    i0 = np.where(m == 0, 0, np.where(m == last, n_tiles - 1, p))
    i1 = np.where(m == 0, 0, np.where(m == last, n_tiles - 1,
                                      np.minimum(p + 1, n_tiles - 1)))
    r = (pos - (2 * p + 1) * half).astype(np.float32)
    denom = np.float32(2 * half - 1)
    w = np.where(interior, (denom - r) / denom, np.float32(1.0)).astype(np.float32)
    ci0 = i0.astype(np.int32) * NBINS
    ci1 = i1.astype(np.int32) * NBINS
    cpk = (ci0 | (ci1 << 16)).astype(np.int32)
    return cpk, w


def _hist_lut_body(x_hbm, lut_hbm, inb0, inb1, hist, lutb, sem0, sem1):
    cid = lax.axis_index("c")
    sid = lax.axis_index("s")
    wid = sid * 2 + cid
    ones = jnp.full((16,), 1.0, jnp.float32)
    iota_f = lax.iota(jnp.int32, 16).astype(jnp.float32)
    last_idx = jnp.full((16,), 15, jnp.int32)

    def per_tilerow(t, _):
        tr = wid * TROW_PER_TEC + t
        base = tr * TROW_PIX
        d0 = pltpu.async_copy(
            x_hbm.at[pl.ds(base, K1_CHUNK)], inb0, sem0)
        d1 = pltpu.async_copy(
            x_hbm.at[pl.ds(base + K1_CHUNK, K1_CHUNK)], inb1, sem1)

        def zero(k, _c):
            for u in range(4):
                hist[pl.ds(k * 64 + u * 16, 16)] = jnp.zeros((16,), jnp.float32)
            return 0
        lax.fori_loop(0, LUT_PER_TROW // 64, zero, 0)

        def scat_chunk(inb):
            def blk(bk, _c):
                i = bk % 8
                ib = i * NBINS
                for u in range(4):
                    x = inb[pl.ds(bk * 64 + u * 16, 16)]
                    bins = jnp.clip((x * 256.0).astype(jnp.int32), 0, 255)
                    plsc.addupdate_scatter(hist, [bins + ib], ones)
                return 0
            lax.fori_loop(0, K1_CHUNK // 64, blk, 0)

        d0.wait()
        scat_chunk(inb0)
        d1.wait()
        scat_chunk(inb1)

        def per_tile(i, _c):
            hbase = i * NBINS

            def clip_sum(k, acc):
                h = jnp.minimum(hist[pl.ds(hbase + k * 16, 16)], MAXV)
                hist[pl.ds(hbase + k * 16, 16)] = h
                return acc + h
            accv = lax.fori_loop(0, 16, clip_sum,
                                 jnp.zeros((16,), jnp.float32))
            clipped = float(PIXELS) - jnp.sum(accv)
            q = (clipped * (1.0 / NBINS)).astype(jnp.int32).astype(jnp.float32)
            residual = clipped - q * float(NBINS)

            def cdf(k, carry):
                h = hist[pl.ds(hbase + k * 16, 16)]
                ind = jnp.where(iota_f + k.astype(jnp.float32) * 16.0 < residual,
                                1.0, 0.0)
                h2 = h + q + ind
                cs = plsc.cumsum(h2)
                csc = cs + carry
                lv = jnp.clip(csc * LUT_SCALE, 0.0, 255.0)
                lutb[pl.ds(hbase + k * 16, 16)] = (
                    lv.astype(jnp.int32).astype(jnp.float32))
                return lax.gather(
                    csc, last_idx[:, None],
                    lax.GatherDimensionNumbers(
                        offset_dims=(), collapsed_slice_dims=(0,),
                        start_index_map=(0,)),
                    (1,), mode=lax.GatherScatterMode.PROMISE_IN_BOUNDS)
            lax.fori_loop(0, 16, cdf, jnp.zeros((16,), jnp.float32))
            return 0
        lax.fori_loop(0, GW, per_tile, 0)

        pltpu.sync_copy(lutb, lut_hbm.at[pl.ds(tr * LUT_PER_TROW, LUT_PER_TROW)])
        return 0
    lax.fori_loop(0, TROW_PER_TEC, per_tilerow, 0)


def _interp_body(x_hbm, lut_hbm, cpk_h, wx_h, out_hbm,
                 inb0, inb1, outb0, outb1, lutv, pairb, cpkv, wxv,
                 sem_i0, sem_i1, sem_o0, sem_o1):
    cid = lax.axis_index("c")
    sid = lax.axis_index("s")
    wid = sid * 2 + cid
    pltpu.sync_copy(cpk_h, cpkv)
    pltpu.sync_copy(wx_h, wxv)
    ins = (inb0, inb1)
    outs = (outb0, outb1)
    sis = (sem_i0, sem_i1)
    sos = (sem_o0, sem_o1)

    def per_half(hq, _):
        hw = wid * HALF_PER_TEC + hq
        img = hw // 2
        half = hw % 2
        jbase = half * 3
        pixbase = img * IMG_PIX + half * HALF_PIX
        d_in = [None, None]
        d_out = [None, None]
        d_in[0] = pltpu.async_copy(
            x_hbm.at[pl.ds(pixbase, K2_CHUNK)], ins[0], sis[0])
        pltpu.sync_copy(
            lut_hbm.at[pl.ds(img * GH * LUT_PER_TROW + jbase * LUT_PER_TROW,
                             LUT_BLK)], lutv)

        for ch in range(K2_NCHUNK):
            bi = ch % 2
            if ch + 1 < K2_NCHUNK:
                d_in[1 - bi] = pltpu.async_copy(
                    x_hbm.at[pl.ds(pixbase + (ch + 1) * K2_CHUNK, K2_CHUNK)],
                    ins[1 - bi], sis[1 - bi])
            m = half * 8 + ch
            p = jnp.clip((m - 1) // 2, 0, GH - 2)
            j0 = jnp.where(m == 0, 0, jnp.where(m == 15, GH - 1, p))
            j1 = jnp.where(m == 0, 0,
                           jnp.where(m == 15, GH - 1,
                                     jnp.minimum(p + 1, GH - 1)))
            interior = jnp.logical_and(m > 0, m < 15)
            ro0 = (j0 - jbase) * LUT_PER_TROW
            ro1 = (j1 - jbase) * LUT_PER_TROW
            rr0 = (half * 256 + ch * 32 - (2 * p + 1) * HALF).astype(jnp.float32)

            def mkpair(k, _c):
                for u in range(4):
                    o = k * 64 + u * 16
                    a = lutv[pl.ds(ro0 + o, 16)]
                    b = lutv[pl.ds(ro1 + o, 16)]
                    w = plsc.bitcast(
                        plsc.pack(a, b, format=plsc.PackFormat.INTERLEAVED),
                        jnp.int32)
                    pairb[pl.ds(o, 16)] = w
                return 0
            lax.fori_loop(0, LUT_PER_TROW // 64, mkpair, 0)

            if d_out[bi] is not None:
                d_out[bi].wait()
            d_in[bi].wait()
            outb = outs[bi]
            inb = ins[bi]

            def per_row(rl, _c):
                rlf = rl.astype(jnp.float32)
                wy = jnp.where(interior, (63.0 - (rr0 + rlf)) * (1.0 / 63.0),
                               1.0)
                wy1 = 1.0 - wy
                rowb = rl * W

                @plsc.parallel_loop(0, W // 16, unroll=2)
                def pg(g):
                    x = inb[pl.ds(rowb + g * 16, 16)]
                    v = (x * 255.0).astype(jnp.int32)
                    cpk = cpkv[pl.ds(g * 16, 16)]
                    c0 = jnp.bitwise_and(cpk, 0xFFFF)
                    c1 = lax.shift_right_logical(cpk, 16)
                    pw0 = plsc.load_gather(pairb, [v + c0])
                    pw1 = plsc.load_gather(pairb, [v + c1])
                    o00, o10 = plsc.unpack(
                        plsc.bitcast(pw0, jnp.bfloat16),
                        format=plsc.PackFormat.INTERLEAVED)
                    o01, o11 = plsc.unpack(
                        plsc.bitcast(pw1, jnp.bfloat16),
                        format=plsc.PackFormat.INTERLEAVED)
                    wx = wxv[pl.ds(g * 16, 16)]
                    wx1 = 1.0 - wx
                    m0 = wy * o00 + wy1 * o10
                    m1 = wy * o01 + wy1 * o11
                    outb[pl.ds(rowb + g * 16, 16)] = (
                        (wx * m0 + wx1 * m1) * (1.0 / 255.0))
                return 0
            lax.fori_loop(0, K2_CHUNK // W, per_row, 0)

            d_out[bi] = pltpu.async_copy(
                outb, out_hbm.at[pl.ds(pixbase + ch * K2_CHUNK, K2_CHUNK)],
                sos[bi])
        d_out[0].wait()
        d_out[1].wait()
        return 0
    lax.fori_loop(0, HALF_PER_TEC, per_half, 0)


_hist_lut = pl.kernel(
    _hist_lut_body,
    out_type=jax.ShapeDtypeStruct((LUT_TOTAL,), jnp.float32),
    mesh=_MESH,
    compiler_params=_SC_PARAMS,
    scratch_types=[
        pltpu.VMEM((K1_CHUNK,), jnp.float32),
        pltpu.VMEM((K1_CHUNK,), jnp.float32),
        pltpu.VMEM((LUT_PER_TROW,), jnp.float32),
        pltpu.VMEM((LUT_PER_TROW,), jnp.float32),
        pltpu.SemaphoreType.DMA,
        pltpu.SemaphoreType.DMA,
    ],
)

_interp = pl.kernel(
    _interp_body,
    out_type=jax.ShapeDtypeStruct((NIMG * IMG_PIX,), jnp.float32),
    mesh=_MESH,
    compiler_params=_SC_PARAMS,
    scratch_types=[
        pltpu.VMEM((K2_CHUNK,), jnp.float32),
        pltpu.VMEM((K2_CHUNK,), jnp.float32),
        pltpu.VMEM((K2_CHUNK,), jnp.float32),
        pltpu.VMEM((K2_CHUNK,), jnp.float32),
        pltpu.VMEM((LUT_BLK,), jnp.float32),
        pltpu.VMEM((LUT_PER_TROW,), jnp.int32),
        pltpu.VMEM((W,), jnp.int32),
        pltpu.VMEM((W,), jnp.float32),
        pltpu.SemaphoreType.DMA,
        pltpu.SemaphoreType.DMA,
        pltpu.SemaphoreType.DMA,
        pltpu.SemaphoreType.DMA,
    ],
)


def kernel(input):
    x_flat = input.reshape(-1)
    cpk, wx = _axis_tables(W, HALF, GW)
    luts = _hist_lut(x_flat)
    out = _interp(x_flat, luts, jnp.asarray(cpk), jnp.asarray(wx))
    return out.reshape(input.shape)


# speedup vs baseline: 2178.0231x; 1.7528x over previous
import functools
import math

import jax
import jax.numpy as jnp
import numpy as np
from jax import lax
from jax.experimental import pallas as pl
from jax.experimental.pallas import tpu as pltpu
from jax.experimental.pallas import tpu_sc as plsc

B, C, H, W = 16, 3, 512, 512
GH = GW = 8
TS = 64
HALF = TS // 2
NBINS = 256
PIXELS = TS * TS
MAXV = 640.0
LUT_SCALE = (NBINS - 1) / PIXELS

NIMG = B * C
IMG_PIX = H * W
TROW_PIX = TS * W
NTROW = NIMG * GH
TROW_PER_TEC = NTROW // 32
LUT_PER_TROW = GW * NBINS
LUT_TOTAL = NTROW * LUT_PER_TROW

K1_CHUNK = TROW_PIX // 2
NHALF = NIMG * 2
HALF_PER_TEC = NHALF // 32
HALF_PIX = IMG_PIX // 2
K2_CHUNK = 32 * W
K2_NCHUNK = HALF_PIX // K2_CHUNK
LUT_BLK = 5 * LUT_PER_TROW

_MESH = plsc.VectorSubcoreMesh(
    core_axis_name="c", subcore_axis_name="s", num_cores=2, num_subcores=16)
_SC_PARAMS = pltpu.CompilerParams(needs_layout_passes=False)


def _axis_tables(n_pix, half, n_tiles):
    pos = np.arange(n_pix)
    m = pos // half
    last = 2 * n_tiles - 1
    interior = (m > 0) & (m < last)
    p = np.clip((m - 1) // 2, 0, n_tiles - 2)
    i0 = np.where(m == 0, 0, np.where(m == last, n_tiles - 1, p))
    i1 = np.where(m == 0, 0, np.where(m == last, n_tiles - 1,
                                      np.minimum(p + 1, n_tiles - 1)))
    r = (pos - (2 * p + 1) * half).astype(np.float32)
    denom = np.float32(2 * half - 1)
    w = np.where(interior, (denom - r) / denom, np.float32(1.0)).astype(np.float32)
    ci0 = i0.astype(np.int32) * NBINS
    ci1 = i1.astype(np.int32) * NBINS
    cpk = (ci0 | (ci1 << 16)).astype(np.int32)
    return cpk, w


def _hist_lut_body(x_hbm, lut_hbm, inb0, inb1, hist, lutb, sem0, sem1, semw):
    cid = lax.axis_index("c")
    sid = lax.axis_index("s")
    wid = sid * 2 + cid
    ones = jnp.full((16,), 1.0, jnp.float32)
    iota_f = lax.iota(jnp.int32, 16).astype(jnp.float32)
    last_idx = jnp.full((16,), 15, jnp.int32)

    base0 = wid * TROW_PER_TEC * TROW_PIX
    pltpu.async_copy(x_hbm.at[pl.ds(base0, K1_CHUNK)], inb0, sem0)
    pltpu.async_copy(x_hbm.at[pl.ds(base0 + K1_CHUNK, K1_CHUNK)], inb1, sem1)

    def per_tilerow(t, _):
        tr = wid * TROW_PER_TEC + t
        base = tr * TROW_PIX

        def zero(k, _c):
            for u in range(4):
                hist[pl.ds(k * 64 + u * 16, 16)] = jnp.zeros((16,), jnp.float32)
            return 0
        lax.fori_loop(0, LUT_PER_TROW // 64, zero, 0)

        def scat_chunk(inb):
            @plsc.parallel_loop(0, K1_CHUNK // 64, unroll=2)
            def blk(bk):
                i = bk % 8
                ib = i * NBINS
                for u in range(4):
                    x = inb[pl.ds(bk * 64 + u * 16, 16)]
                    bins = jnp.clip((x * 256.0).astype(jnp.int32), 0, 255)
                    plsc.addupdate_scatter(hist, [bins + ib], ones)

        pltpu.make_async_copy(x_hbm.at[pl.ds(base, K1_CHUNK)], inb0, sem0).wait()
        scat_chunk(inb0)
        pltpu.make_async_copy(
            x_hbm.at[pl.ds(base + K1_CHUNK, K1_CHUNK)], inb1, sem1).wait()
        scat_chunk(inb1)

        @pl.when(t < TROW_PER_TEC - 1)
        def _prefetch():
            nbase = base + TROW_PIX
            pltpu.async_copy(x_hbm.at[pl.ds(nbase, K1_CHUNK)], inb0, sem0)
            pltpu.async_copy(
                x_hbm.at[pl.ds(nbase + K1_CHUNK, K1_CHUNK)], inb1, sem1)

        @pl.when(t > 0)
        def _drain_lut():
            pltpu.make_async_copy(
                lutb, lut_hbm.at[pl.ds(tr * LUT_PER_TROW, LUT_PER_TROW)],
                semw).wait()

        def per_tile(i, _c):
            hbase = i * NBINS

            def clip_sum(k, acc):
                h = jnp.minimum(hist[pl.ds(hbase + k * 16, 16)], MAXV)
                hist[pl.ds(hbase + k * 16, 16)] = h
                return acc + h
            accv = lax.fori_loop(0, 16, clip_sum,
                                 jnp.zeros((16,), jnp.float32))
            clipped = float(PIXELS) - jnp.sum(accv)
            q = (clipped * (1.0 / NBINS)).astype(jnp.int32).astype(jnp.float32)
            residual = clipped - q * float(NBINS)

            def cdf(k, carry):
                h = hist[pl.ds(hbase + k * 16, 16)]
                ind = jnp.where(iota_f + k.astype(jnp.float32) * 16.0 < residual,
                                1.0, 0.0)
                h2 = h + q + ind
                cs = plsc.cumsum(h2)
                csc = cs + carry
                lv = jnp.clip(csc * LUT_SCALE, 0.0, 255.0)
                lutb[pl.ds(hbase + k * 16, 16)] = (
                    lv.astype(jnp.int32).astype(jnp.float32))
                return lax.gather(
                    csc, last_idx[:, None],
                    lax.GatherDimensionNumbers(
                        offset_dims=(), collapsed_slice_dims=(0,),
                        start_index_map=(0,)),
                    (1,), mode=lax.GatherScatterMode.PROMISE_IN_BOUNDS)
            lax.fori_loop(0, 16, cdf, jnp.zeros((16,), jnp.float32))
            return 0
        lax.fori_loop(0, GW, per_tile, 0)

        pltpu.async_copy(
            lutb, lut_hbm.at[pl.ds(tr * LUT_PER_TROW, LUT_PER_TROW)], semw)
        return 0
    lax.fori_loop(0, TROW_PER_TEC, per_tilerow, 0)
    last_tr = wid * TROW_PER_TEC + TROW_PER_TEC - 1
    pltpu.make_async_copy(
        lutb, lut_hbm.at[pl.ds(last_tr * LUT_PER_TROW, LUT_PER_TROW)],
        semw).wait()


def _interp_body(x_hbm, lut_hbm, cpk_h, wx_h, out_hbm,
                 inb0, inb1, outb0, outb1, lutv, pairb, cpkv, wxv,
                 sem_i0, sem_i1, sem_o0, sem_o1):
    cid = lax.axis_index("c")
    sid = lax.axis_index("s")
    wid = sid * 2 + cid
    pltpu.sync_copy(cpk_h, cpkv)
    pltpu.sync_copy(wx_h, wxv)
    ins = (inb0, inb1)
    outs = (outb0, outb1)
    sis = (sem_i0, sem_i1)
    sos = (sem_o0, sem_o1)

    def per_half(hq, _):
        hw = wid * HALF_PER_TEC + hq
        img = hw // 2
        half = hw % 2
        jbase = half * 3
        pixbase = img * IMG_PIX + half * HALF_PIX
        d_in = [None, None]
        d_out = [None, None]
        d_in[0] = pltpu.async_copy(
            x_hbm.at[pl.ds(pixbase, K2_CHUNK)], ins[0], sis[0])
        pltpu.sync_copy(
            lut_hbm.at[pl.ds(img * GH * LUT_PER_TROW + jbase * LUT_PER_TROW,
                             LUT_BLK)], lutv)

        for ch in range(K2_NCHUNK):
            bi = ch % 2
            if ch + 1 < K2_NCHUNK:
                d_in[1 - bi] = pltpu.async_copy(
                    x_hbm.at[pl.ds(pixbase + (ch + 1) * K2_CHUNK, K2_CHUNK)],
                    ins[1 - bi], sis[1 - bi])
            m = half * 8 + ch
            p = jnp.clip((m - 1) // 2, 0, GH - 2)
            j0 = jnp.where(m == 0, 0, jnp.where(m == 15, GH - 1, p))
            j1 = jnp.where(m == 0, 0,
                           jnp.where(m == 15, GH - 1,
                                     jnp.minimum(p + 1, GH - 1)))
            interior = jnp.logical_and(m > 0, m < 15)
            ro0 = (j0 - jbase) * LUT_PER_TROW
            ro1 = (j1 - jbase) * LUT_PER_TROW
            rr0 = (half * 256 + ch * 32 - (2 * p + 1) * HALF).astype(jnp.float32)

            def mkpair(k, _c):
                for u in range(4):
                    o = k * 64 + u * 16
                    a = lutv[pl.ds(ro0 + o, 16)]
                    b = lutv[pl.ds(ro1 + o, 16)]
                    w = plsc.bitcast(
                        plsc.pack(a, b, format=plsc.PackFormat.INTERLEAVED),
                        jnp.int32)
                    pairb[pl.ds(o, 16)] = w
                return 0
            lax.fori_loop(0, LUT_PER_TROW // 64, mkpair, 0)

            if d_out[bi] is not None:
                d_out[bi].wait()
            d_in[bi].wait()
            outb = outs[bi]
            inb = ins[bi]

            def per_row(rl, _c):
                rlf = rl.astype(jnp.float32)
                wy = jnp.where(interior, (63.0 - (rr0 + rlf)) * (1.0 / 63.0),
                               1.0)
                wy1 = 1.0 - wy
                rowb = rl * W

                @plsc.parallel_loop(0, W // 16, unroll=2)
                def pg(g):
                    x = inb[pl.ds(rowb + g * 16, 16)]
                    v = (x * 255.0).astype(jnp.int32)
                    cpk = cpkv[pl.ds(g * 16, 16)]
                    c0 = jnp.bitwise_and(cpk, 0xFFFF)
                    c1 = lax.shift_right_logical(cpk, 16)
                    pw0 = plsc.load_gather(pairb, [v + c0])
                    pw1 = plsc.load_gather(pairb, [v + c1])
                    o00, o10 = plsc.unpack(
                        plsc.bitcast(pw0, jnp.bfloat16),
                        format=plsc.PackFormat.INTERLEAVED)
                    o01, o11 = plsc.unpack(
                        plsc.bitcast(pw1, jnp.bfloat16),
                        format=plsc.PackFormat.INTERLEAVED)
                    wx = wxv[pl.ds(g * 16, 16)]
                    wx1 = 1.0 - wx
                    m0 = wy * o00 + wy1 * o10
                    m1 = wy * o01 + wy1 * o11
                    outb[pl.ds(rowb + g * 16, 16)] = (
                        (wx * m0 + wx1 * m1) * (1.0 / 255.0))
                return 0
            lax.fori_loop(0, K2_CHUNK // W, per_row, 0)

            d_out[bi] = pltpu.async_copy(
                outb, out_hbm.at[pl.ds(pixbase + ch * K2_CHUNK, K2_CHUNK)],
                sos[bi])
        d_out[0].wait()
        d_out[1].wait()
        return 0
    lax.fori_loop(0, HALF_PER_TEC, per_half, 0)


_hist_lut = pl.kernel(
    _hist_lut_body,
    out_type=jax.ShapeDtypeStruct((LUT_TOTAL,), jnp.float32),
    mesh=_MESH,
    compiler_params=_SC_PARAMS,
    scratch_types=[
        pltpu.VMEM((K1_CHUNK,), jnp.float32),
        pltpu.VMEM((K1_CHUNK,), jnp.float32),
        pltpu.VMEM((LUT_PER_TROW,), jnp.float32),
        pltpu.VMEM((LUT_PER_TROW,), jnp.float32),
        pltpu.SemaphoreType.DMA,
        pltpu.SemaphoreType.DMA,
        pltpu.SemaphoreType.DMA,
    ],
)

_interp = pl.kernel(
    _interp_body,
    out_type=jax.ShapeDtypeStruct((NIMG * IMG_PIX,), jnp.float32),
    mesh=_MESH,
    compiler_params=_SC_PARAMS,
    scratch_types=[
        pltpu.VMEM((K2_CHUNK,), jnp.float32),
        pltpu.VMEM((K2_CHUNK,), jnp.float32),
        pltpu.VMEM((K2_CHUNK,), jnp.float32),
        pltpu.VMEM((K2_CHUNK,), jnp.float32),
        pltpu.VMEM((LUT_BLK,), jnp.float32),
        pltpu.VMEM((LUT_PER_TROW,), jnp.int32),
        pltpu.VMEM((W,), jnp.int32),
        pltpu.VMEM((W,), jnp.float32),
        pltpu.SemaphoreType.DMA,
        pltpu.SemaphoreType.DMA,
        pltpu.SemaphoreType.DMA,
        pltpu.SemaphoreType.DMA,
    ],
)


def kernel(input):
    x_flat = input.reshape(-1)
    cpk, wx = _axis_tables(W, HALF, GW)
    luts = _hist_lut(x_flat)
    out = _interp(x_flat, luts, jnp.asarray(cpk), jnp.asarray(wx))
    return out.reshape(input.shape)


# speedup vs baseline: 2196.2065x; 1.0083x over previous
import functools
import math

import jax
import jax.numpy as jnp
import numpy as np
from jax import lax
from jax.experimental import pallas as pl
from jax.experimental.pallas import tpu as pltpu
from jax.experimental.pallas import tpu_sc as plsc

B, C, H, W = 16, 3, 512, 512
GH = GW = 8
TS = 64
HALF = TS // 2
NBINS = 256
PIXELS = TS * TS
MAXV = 640.0
LUT_SCALE = (NBINS - 1) / PIXELS

NIMG = B * C
IMG_PIX = H * W
TROW_PIX = TS * W
NTROW = NIMG * GH
TROW_PER_TEC = NTROW // 32
LUT_PER_TROW = GW * NBINS
LUT_TOTAL = NTROW * LUT_PER_TROW

K1_CHUNK = TROW_PIX // 2
NHALF = NIMG * 2
HALF_PER_TEC = NHALF // 32
HALF_PIX = IMG_PIX // 2
K2_CHUNK = 32 * W
K2_NCHUNK = HALF_PIX // K2_CHUNK
LUT_BLK = 5 * LUT_PER_TROW

_MESH = plsc.VectorSubcoreMesh(
    core_axis_name="c", subcore_axis_name="s", num_cores=2, num_subcores=16)
_SC_PARAMS = pltpu.CompilerParams(needs_layout_passes=False)


def _axis_tables(n_pix, half, n_tiles):
    pos = np.arange(n_pix)
    m = pos // half
    last = 2 * n_tiles - 1
    interior = (m > 0) & (m < last)
    p = np.clip((m - 1) // 2, 0, n_tiles - 2)
    i0 = np.where(m == 0, 0, np.where(m == last, n_tiles - 1, p))
    i1 = np.where(m == 0, 0, np.where(m == last, n_tiles - 1,
                                      np.minimum(p + 1, n_tiles - 1)))
    r = (pos - (2 * p + 1) * half).astype(np.float32)
    denom = np.float32(2 * half - 1)
    w = np.where(interior, (denom - r) / denom, np.float32(1.0)).astype(np.float32)
    ci0 = i0.astype(np.int32) * NBINS
    ci1 = i1.astype(np.int32) * NBINS
    cpk = (ci0 | (ci1 << 16)).astype(np.int32)
    return cpk, w


def _hist_lut_body(x_hbm, lut_hbm, inb0, inb1, hist, lutb, sem0, sem1, semw):
    cid = lax.axis_index("c")
    sid = lax.axis_index("s")
    wid = sid * 2 + cid
    ones = jnp.full((16,), 1.0, jnp.float32)
    iota_f = lax.iota(jnp.int32, 16).astype(jnp.float32)
    last_idx = jnp.full((16,), 15, jnp.int32)

    base0 = wid * TROW_PER_TEC * TROW_PIX
    pltpu.async_copy(x_hbm.at[pl.ds(base0, K1_CHUNK)], inb0, sem0)
    pltpu.async_copy(x_hbm.at[pl.ds(base0 + K1_CHUNK, K1_CHUNK)], inb1, sem1)

    def per_tilerow(t, _):
        tr = wid * TROW_PER_TEC + t
        base = tr * TROW_PIX

        def zero(k, _c):
            for u in range(4):
                hist[pl.ds(k * 64 + u * 16, 16)] = jnp.zeros((16,), jnp.float32)
            return 0
        lax.fori_loop(0, LUT_PER_TROW // 64, zero, 0)

        def scat_chunk(inb):
            @plsc.parallel_loop(0, K1_CHUNK // 64, unroll=2)
            def blk(bk):
                i = bk % 8
                ib = i * NBINS
                for u in range(4):
                    x = inb[pl.ds(bk * 64 + u * 16, 16)]
                    bins = jnp.clip((x * 256.0).astype(jnp.int32), 0, 255)
                    plsc.addupdate_scatter(hist, [bins + ib], ones)

        pltpu.make_async_copy(x_hbm.at[pl.ds(base, K1_CHUNK)], inb0, sem0).wait()
        scat_chunk(inb0)
        pltpu.make_async_copy(
            x_hbm.at[pl.ds(base + K1_CHUNK, K1_CHUNK)], inb1, sem1).wait()
        scat_chunk(inb1)

        @pl.when(t < TROW_PER_TEC - 1)
        def _prefetch():
            nbase = base + TROW_PIX
            pltpu.async_copy(x_hbm.at[pl.ds(nbase, K1_CHUNK)], inb0, sem0)
            pltpu.async_copy(
                x_hbm.at[pl.ds(nbase + K1_CHUNK, K1_CHUNK)], inb1, sem1)

        @pl.when(t > 0)
        def _drain_lut():
            pltpu.make_async_copy(
                lutb, lut_hbm.at[pl.ds(tr * LUT_PER_TROW, LUT_PER_TROW)],
                semw).wait()

        def per_tile(i, _c):
            hbase = i * NBINS

            def clip_sum(k, acc):
                h = jnp.minimum(hist[pl.ds(hbase + k * 16, 16)], MAXV)
                hist[pl.ds(hbase + k * 16, 16)] = h
                return acc + h
            accv = lax.fori_loop(0, 16, clip_sum,
                                 jnp.zeros((16,), jnp.float32))
            clipped = float(PIXELS) - jnp.sum(accv)
            q = (clipped * (1.0 / NBINS)).astype(jnp.int32).astype(jnp.float32)
            residual = clipped - q * float(NBINS)

            def cdf(k, carry):
                h = hist[pl.ds(hbase + k * 16, 16)]
                ind = jnp.where(iota_f + k.astype(jnp.float32) * 16.0 < residual,
                                1.0, 0.0)
                h2 = h + q + ind
                cs = plsc.cumsum(h2)
                csc = cs + carry
                lv = jnp.clip(csc * LUT_SCALE, 0.0, 255.0)
                lutb[pl.ds(hbase + k * 16, 16)] = (
                    lv.astype(jnp.int32).astype(jnp.float32))
                return lax.gather(
                    csc, last_idx[:, None],
                    lax.GatherDimensionNumbers(
                        offset_dims=(), collapsed_slice_dims=(0,),
                        start_index_map=(0,)),
                    (1,), mode=lax.GatherScatterMode.PROMISE_IN_BOUNDS)
            lax.fori_loop(0, 16, cdf, jnp.zeros((16,), jnp.float32))
            return 0
        lax.fori_loop(0, GW, per_tile, 0)

        pltpu.async_copy(
            lutb, lut_hbm.at[pl.ds(tr * LUT_PER_TROW, LUT_PER_TROW)], semw)
        return 0
    lax.fori_loop(0, TROW_PER_TEC, per_tilerow, 0)
    last_tr = wid * TROW_PER_TEC + TROW_PER_TEC - 1
    pltpu.make_async_copy(
        lutb, lut_hbm.at[pl.ds(last_tr * LUT_PER_TROW, LUT_PER_TROW)],
        semw).wait()


def _interp_body(x_hbm, lut_hbm, cpk_h, wx_h, out_hbm,
                 inb0, inb1, outb0, outb1, lutv, pairb, cpkv, wxv,
                 sem_i0, sem_i1, sem_o0, sem_o1, sem_l):
    cid = lax.axis_index("c")
    sid = lax.axis_index("s")
    wid = sid * 2 + cid
    pltpu.sync_copy(cpk_h, cpkv)
    pltpu.sync_copy(wx_h, wxv)
    ins = (inb0, inb1)
    outs = (outb0, outb1)
    sis = (sem_i0, sem_i1)
    sos = (sem_o0, sem_o1)

    def per_half(hq, _):
        hw = wid * HALF_PER_TEC + hq
        img = hw // 2
        half = hw % 2
        jbase = half * 3
        pixbase = img * IMG_PIX + half * HALF_PIX
        d_in = [None, None]
        d_out = [None, None]
        d_in[0] = pltpu.async_copy(
            x_hbm.at[pl.ds(pixbase, K2_CHUNK)], ins[0], sis[0])
        d_lut = pltpu.async_copy(
            lut_hbm.at[pl.ds(img * GH * LUT_PER_TROW + jbase * LUT_PER_TROW,
                             LUT_BLK)], lutv, sem_l)

        for ch in range(K2_NCHUNK):
            bi = ch % 2
            if ch + 1 < K2_NCHUNK:
                d_in[1 - bi] = pltpu.async_copy(
                    x_hbm.at[pl.ds(pixbase + (ch + 1) * K2_CHUNK, K2_CHUNK)],
                    ins[1 - bi], sis[1 - bi])
            m = half * 8 + ch
            p = jnp.clip((m - 1) // 2, 0, GH - 2)
            j0 = jnp.where(m == 0, 0, jnp.where(m == 15, GH - 1, p))
            j1 = jnp.where(m == 0, 0,
                           jnp.where(m == 15, GH - 1,
                                     jnp.minimum(p + 1, GH - 1)))
            interior = jnp.logical_and(m > 0, m < 15)
            ro0 = (j0 - jbase) * LUT_PER_TROW
            ro1 = (j1 - jbase) * LUT_PER_TROW
            rr0 = (half * 256 + ch * 32 - (2 * p + 1) * HALF).astype(jnp.float32)

            if ch == 0:
                d_lut.wait()

            @plsc.parallel_loop(0, LUT_PER_TROW // 64, unroll=2)
            def mkpair(k):
                for u in range(4):
                    o = k * 64 + u * 16
                    a = lutv[pl.ds(ro0 + o, 16)]
                    b = lutv[pl.ds(ro1 + o, 16)]
                    w = plsc.bitcast(
                        plsc.pack(a, b, format=plsc.PackFormat.INTERLEAVED),
                        jnp.int32)
                    pairb[pl.ds(o, 16)] = w

            if d_out[bi] is not None:
                d_out[bi].wait()
            d_in[bi].wait()
            outb = outs[bi]
            inb = ins[bi]

            def per_row(rl, _c):
                rlf = rl.astype(jnp.float32)
                wy = jnp.where(interior, (63.0 - (rr0 + rlf)) * (1.0 / 63.0),
                               1.0)
                wy1 = 1.0 - wy
                rowb = rl * W

                @plsc.parallel_loop(0, W // 16, unroll=4)
                def pg(g):
                    x = inb[pl.ds(rowb + g * 16, 16)]
                    v = (x * 255.0).astype(jnp.int32)
                    cpk = cpkv[pl.ds(g * 16, 16)]
                    c0 = jnp.bitwise_and(cpk, 0xFFFF)
                    c1 = lax.shift_right_logical(cpk, 16)
                    pw0 = plsc.load_gather(pairb, [v + c0])
                    pw1 = plsc.load_gather(pairb, [v + c1])
                    o00, o10 = plsc.unpack(
                        plsc.bitcast(pw0, jnp.bfloat16),
                        format=plsc.PackFormat.INTERLEAVED)
                    o01, o11 = plsc.unpack(
                        plsc.bitcast(pw1, jnp.bfloat16),
                        format=plsc.PackFormat.INTERLEAVED)
                    wx = wxv[pl.ds(g * 16, 16)]
                    wx1 = 1.0 - wx
                    m0 = wy * o00 + wy1 * o10
                    m1 = wy * o01 + wy1 * o11
                    outb[pl.ds(rowb + g * 16, 16)] = (
                        (wx * m0 + wx1 * m1) * (1.0 / 255.0))
                return 0
            lax.fori_loop(0, K2_CHUNK // W, per_row, 0)

            d_out[bi] = pltpu.async_copy(
                outb, out_hbm.at[pl.ds(pixbase + ch * K2_CHUNK, K2_CHUNK)],
                sos[bi])
        d_out[0].wait()
        d_out[1].wait()
        return 0
    lax.fori_loop(0, HALF_PER_TEC, per_half, 0)


_hist_lut = pl.kernel(
    _hist_lut_body,
    out_type=jax.ShapeDtypeStruct((LUT_TOTAL,), jnp.float32),
    mesh=_MESH,
    compiler_params=_SC_PARAMS,
    scratch_types=[
        pltpu.VMEM((K1_CHUNK,), jnp.float32),
        pltpu.VMEM((K1_CHUNK,), jnp.float32),
        pltpu.VMEM((LUT_PER_TROW,), jnp.float32),
        pltpu.VMEM((LUT_PER_TROW,), jnp.float32),
        pltpu.SemaphoreType.DMA,
        pltpu.SemaphoreType.DMA,
        pltpu.SemaphoreType.DMA,
    ],
)

_interp = pl.kernel(
    _interp_body,
    out_type=jax.ShapeDtypeStruct((NIMG * IMG_PIX,), jnp.float32),
    mesh=_MESH,
    compiler_params=_SC_PARAMS,
    scratch_types=[
        pltpu.VMEM((K2_CHUNK,), jnp.float32),
        pltpu.VMEM((K2_CHUNK,), jnp.float32),
        pltpu.VMEM((K2_CHUNK,), jnp.float32),
        pltpu.VMEM((K2_CHUNK,), jnp.float32),
        pltpu.VMEM((LUT_BLK,), jnp.float32),
        pltpu.VMEM((LUT_PER_TROW,), jnp.int32),
        pltpu.VMEM((W,), jnp.int32),
        pltpu.VMEM((W,), jnp.float32),
        pltpu.SemaphoreType.DMA,
        pltpu.SemaphoreType.DMA,
        pltpu.SemaphoreType.DMA,
        pltpu.SemaphoreType.DMA,
        pltpu.SemaphoreType.DMA,
    ],
)


def kernel(input):
    x_flat = input.reshape(-1)
    cpk, wx = _axis_tables(W, HALF, GW)
    luts = _hist_lut(x_flat)
    out = _interp(x_flat, luts, jnp.asarray(cpk), jnp.asarray(wx))
    return out.reshape(input.shape)


# speedup vs baseline: 2217.0745x; 1.0095x over previous
import jax
import jax.numpy as jnp
import numpy as np
from jax import lax
from jax.experimental import pallas as pl
from jax.experimental.pallas import tpu as pltpu
from jax.experimental.pallas import tpu_sc as plsc

B, C, H, W = 16, 3, 512, 512
GH = GW = 8
TS = 64
HALF = TS // 2
NBINS = 256
PIXELS = TS * TS
MAXV = 640.0
LUT_SCALE = (NBINS - 1) / PIXELS

NIMG = B * C
IMG_PER_SC = NIMG // 2
IMG_PIX = H * W
TROW_PIX = TS * W
TROW_PER_TEC = IMG_PER_SC * GH // 16
LUT_PER_TROW = GW * NBINS
LUT_SC = IMG_PER_SC * GH * LUT_PER_TROW

K_CHUNK = 32 * W
NCHUNK_HALF = IMG_PIX // 2 // K_CHUNK
HALF_PER_TEC = IMG_PER_SC * 2 // 16
LUT_BLK = 5 * LUT_PER_TROW

_MESH = plsc.VectorSubcoreMesh(
    core_axis_name="c", subcore_axis_name="s", num_cores=2, num_subcores=16)
_SC_PARAMS = pltpu.CompilerParams(needs_layout_passes=False)


def _axis_tables(n_pix, half, n_tiles):
    pos = np.arange(n_pix)
    m = pos // half
    last = 2 * n_tiles - 1
    interior = (m > 0) & (m < last)
    p = np.clip((m - 1) // 2, 0, n_tiles - 2)
    i0 = np.where(m == 0, 0, np.where(m == last, n_tiles - 1, p))
    i1 = np.where(m == 0, 0, np.where(m == last, n_tiles - 1,
                                      np.minimum(p + 1, n_tiles - 1)))
    r = (pos - (2 * p + 1) * half).astype(np.float32)
    denom = np.float32(2 * half - 1)
    w = np.where(interior, (denom - r) / denom, np.float32(1.0)).astype(np.float32)
    ci0 = i0.astype(np.int32) * NBINS
    ci1 = i1.astype(np.int32) * NBINS
    cpk = (ci0 | (ci1 << 16)).astype(np.int32)
    return cpk, w


def _clahe_body(x_hbm, cpk_h, wx_h, out_hbm,
                inb0, inb1, outb0, outb1, hist, lutb, lutv, pairb, cpkv, wxv,
                lut_sh, sem0, sem1, semw, sem_o0, sem_o1):
    cid = lax.axis_index("c")
    sid = lax.axis_index("s")
    ones = jnp.full((16,), 1.0, jnp.float32)
    iota_f = lax.iota(jnp.int32, 16).astype(jnp.float32)
    last_idx = jnp.full((16,), 15, jnp.int32)

    ltr0 = sid * TROW_PER_TEC
    gbase0 = (cid * IMG_PER_SC * GH + ltr0) * TROW_PIX
    pltpu.async_copy(x_hbm.at[pl.ds(gbase0, K_CHUNK)], inb0, sem0)
    pltpu.async_copy(x_hbm.at[pl.ds(gbase0 + K_CHUNK, K_CHUNK)], inb1, sem1)

    def per_tilerow(t, _):
        ltr = ltr0 + t
        base = (cid * IMG_PER_SC * GH + ltr) * TROW_PIX

        def zero(k, _c):
            for u in range(4):
                hist[pl.ds(k * 64 + u * 16, 16)] = jnp.zeros((16,), jnp.float32)
            return 0
        lax.fori_loop(0, LUT_PER_TROW // 64, zero, 0)

        def scat_chunk(inb):
            @plsc.parallel_loop(0, K_CHUNK // 64, unroll=2)
            def blk(bk):
                i = bk % 8
                ib = i * NBINS
                for u in range(4):
                    x = inb[pl.ds(bk * 64 + u * 16, 16)]
                    bins = jnp.clip((x * 256.0).astype(jnp.int32), 0, 255)
                    plsc.addupdate_scatter(hist, [bins + ib], ones)

        pltpu.make_async_copy(x_hbm.at[pl.ds(base, K_CHUNK)], inb0, sem0).wait()
        scat_chunk(inb0)
        pltpu.make_async_copy(
            x_hbm.at[pl.ds(base + K_CHUNK, K_CHUNK)], inb1, sem1).wait()
        scat_chunk(inb1)

        @pl.when(t < TROW_PER_TEC - 1)
        def _prefetch():
            nbase = base + TROW_PIX
            pltpu.async_copy(x_hbm.at[pl.ds(nbase, K_CHUNK)], inb0, sem0)
            pltpu.async_copy(
                x_hbm.at[pl.ds(nbase + K_CHUNK, K_CHUNK)], inb1, sem1)

        @pl.when(t > 0)
        def _drain_lut():
            pltpu.make_async_copy(
                lutb, lut_sh.at[pl.ds(ltr * LUT_PER_TROW, LUT_PER_TROW)],
                semw).wait()

        def per_tile(i, _c):
            hbase = i * NBINS

            def clip_sum(k, acc):
                h = jnp.minimum(hist[pl.ds(hbase + k * 16, 16)], MAXV)
                hist[pl.ds(hbase + k * 16, 16)] = h
                return acc + h
            accv = lax.fori_loop(0, 16, clip_sum,
                                 jnp.zeros((16,), jnp.float32))
            clipped = float(PIXELS) - jnp.sum(accv)
            q = (clipped * (1.0 / NBINS)).astype(jnp.int32).astype(jnp.float32)
            residual = clipped - q * float(NBINS)

            def cdf(k, carry):
                h = hist[pl.ds(hbase + k * 16, 16)]
                ind = jnp.where(iota_f + k.astype(jnp.float32) * 16.0 < residual,
                                1.0, 0.0)
                h2 = h + q + ind
                cs = plsc.cumsum(h2)
                csc = cs + carry
                lv = jnp.clip(csc * LUT_SCALE, 0.0, 255.0)
                lutb[pl.ds(hbase + k * 16, 16)] = (
                    lv.astype(jnp.int32).astype(jnp.float32))
                return lax.gather(
                    csc, last_idx[:, None],
                    lax.GatherDimensionNumbers(
                        offset_dims=(), collapsed_slice_dims=(0,),
                        start_index_map=(0,)),
                    (1,), mode=lax.GatherScatterMode.PROMISE_IN_BOUNDS)
            lax.fori_loop(0, 16, cdf, jnp.zeros((16,), jnp.float32))
            return 0
        lax.fori_loop(0, GW, per_tile, 0)

        pltpu.async_copy(
            lutb, lut_sh.at[pl.ds(ltr * LUT_PER_TROW, LUT_PER_TROW)], semw)
        return 0
    lax.fori_loop(0, TROW_PER_TEC, per_tilerow, 0)
    last_ltr = ltr0 + TROW_PER_TEC - 1
    pltpu.make_async_copy(
        lutb, lut_sh.at[pl.ds(last_ltr * LUT_PER_TROW, LUT_PER_TROW)],
        semw).wait()

    plsc.subcore_barrier()

    pltpu.sync_copy(cpk_h, cpkv)
    pltpu.sync_copy(wx_h, wxv)
    ins = (inb0, inb1)
    outs = (outb0, outb1)
    sis = (sem0, sem1)
    sos = (sem_o0, sem_o1)

    def per_half(hq, _):
        hw = sid * HALF_PER_TEC + hq
        img_l = hw // 2
        half = hw % 2
        jbase = half * 3
        pixbase = (cid * IMG_PER_SC + img_l) * IMG_PIX + half * (IMG_PIX // 2)
        d_in = [None, None]
        d_out = [None, None]
        d_in[0] = pltpu.async_copy(
            x_hbm.at[pl.ds(pixbase, K_CHUNK)], ins[0], sis[0])
        d_lut = pltpu.async_copy(
            lut_sh.at[pl.ds((img_l * GH + jbase) * LUT_PER_TROW, LUT_BLK)],
            lutv, semw)

        for ch in range(NCHUNK_HALF):
            bi = ch % 2
            if ch + 1 < NCHUNK_HALF:
                d_in[1 - bi] = pltpu.async_copy(
                    x_hbm.at[pl.ds(pixbase + (ch + 1) * K_CHUNK, K_CHUNK)],
                    ins[1 - bi], sis[1 - bi])
            m = half * 8 + ch
            p = jnp.clip((m - 1) // 2, 0, GH - 2)
            j0 = jnp.where(m == 0, 0, jnp.where(m == 15, GH - 1, p))
            j1 = jnp.where(m == 0, 0,
                           jnp.where(m == 15, GH - 1,
                                     jnp.minimum(p + 1, GH - 1)))
            interior = jnp.logical_and(m > 0, m < 15)
            ro0 = (j0 - jbase) * LUT_PER_TROW
            ro1 = (j1 - jbase) * LUT_PER_TROW
            rr0 = (half * 256 + ch * 32 - (2 * p + 1) * HALF).astype(jnp.float32)

            if ch == 0:
                d_lut.wait()

            @plsc.parallel_loop(0, LUT_PER_TROW // 64, unroll=2)
            def mkpair(k):
                for u in range(4):
                    o = k * 64 + u * 16
                    a = lutv[pl.ds(ro0 + o, 16)]
                    b = lutv[pl.ds(ro1 + o, 16)]
                    w = plsc.bitcast(
                        plsc.pack(a, b, format=plsc.PackFormat.INTERLEAVED),
                        jnp.int32)
                    pairb[pl.ds(o, 16)] = w

            if d_out[bi] is not None:
                d_out[bi].wait()
            d_in[bi].wait()
            outb = outs[bi]
            inb = ins[bi]

            def per_row(rl, _c):
                rlf = rl.astype(jnp.float32)
                wy = jnp.where(interior, (63.0 - (rr0 + rlf)) * (1.0 / 63.0),
                               1.0)
                wy1 = 1.0 - wy
                rowb = rl * W

                @plsc.parallel_loop(0, W // 16, unroll=4)
                def pg(g):
                    x = inb[pl.ds(rowb + g * 16, 16)]
                    v = (x * 255.0).astype(jnp.int32)
                    cpk = cpkv[pl.ds(g * 16, 16)]
                    c0 = jnp.bitwise_and(cpk, 0xFFFF)
                    c1 = lax.shift_right_logical(cpk, 16)
                    pw0 = plsc.load_gather(pairb, [v + c0])
                    pw1 = plsc.load_gather(pairb, [v + c1])
                    o00, o10 = plsc.unpack(
                        plsc.bitcast(pw0, jnp.bfloat16),
                        format=plsc.PackFormat.INTERLEAVED)
                    o01, o11 = plsc.unpack(
                        plsc.bitcast(pw1, jnp.bfloat16),
                        format=plsc.PackFormat.INTERLEAVED)
                    wx = wxv[pl.ds(g * 16, 16)]
                    wx1 = 1.0 - wx
                    m0 = wy * o00 + wy1 * o10
                    m1 = wy * o01 + wy1 * o11
                    outb[pl.ds(rowb + g * 16, 16)] = (
                        (wx * m0 + wx1 * m1) * (1.0 / 255.0))
                return 0
            lax.fori_loop(0, K_CHUNK // W, per_row, 0)

            d_out[bi] = pltpu.async_copy(
                outb, out_hbm.at[pl.ds(pixbase + ch * K_CHUNK, K_CHUNK)],
                sos[bi])
        d_out[0].wait()
        d_out[1].wait()
        return 0
    lax.fori_loop(0, HALF_PER_TEC, per_half, 0)


_clahe = pl.kernel(
    _clahe_body,
    out_type=jax.ShapeDtypeStruct((NIMG * IMG_PIX,), jnp.float32),
    mesh=_MESH,
    compiler_params=_SC_PARAMS,
    scratch_types=[
        pltpu.VMEM((K_CHUNK,), jnp.float32),
        pltpu.VMEM((K_CHUNK,), jnp.float32),
        pltpu.VMEM((K_CHUNK,), jnp.float32),
        pltpu.VMEM((K_CHUNK,), jnp.float32),
        pltpu.VMEM((LUT_PER_TROW,), jnp.float32),
        pltpu.VMEM((LUT_PER_TROW,), jnp.float32),
        pltpu.VMEM((LUT_BLK,), jnp.float32),
        pltpu.VMEM((LUT_PER_TROW,), jnp.int32),
        pltpu.VMEM((W,), jnp.int32),
        pltpu.VMEM((W,), jnp.float32),
        pltpu.VMEM_SHARED((LUT_SC,), jnp.float32),
        pltpu.SemaphoreType.DMA,
        pltpu.SemaphoreType.DMA,
        pltpu.SemaphoreType.DMA,
        pltpu.SemaphoreType.DMA,
        pltpu.SemaphoreType.DMA,
    ],
)


def kernel(input):
    x_flat = input.reshape(-1)
    cpk, wx = _axis_tables(W, HALF, GW)
    out = _clahe(x_flat, jnp.asarray(cpk), jnp.asarray(wx))
    return out.reshape(input.shape)


# speedup vs baseline: 2223.2342x; 1.0028x over previous
import jax
import jax.numpy as jnp
import numpy as np
from jax import lax
from jax.experimental import pallas as pl
from jax.experimental.pallas import tpu as pltpu
from jax.experimental.pallas import tpu_sc as plsc

B, C, H, W = 16, 3, 512, 512
GH = GW = 8
TS = 64
HALF = TS // 2
NBINS = 256
PIXELS = TS * TS
MAXV = 640.0
LUT_SCALE = (NBINS - 1) / PIXELS

NIMG = B * C
IMG_PER_SC = NIMG // 2
IMG_PIX = H * W
TROW_PIX = TS * W
TROW_PER_TEC = IMG_PER_SC * GH // 16
LUT_PER_TROW = GW * NBINS
LUT_SC = IMG_PER_SC * GH * LUT_PER_TROW

K_CHUNK = 32 * W
NCHUNK_HALF = IMG_PIX // 2 // K_CHUNK
HALF_PER_TEC = IMG_PER_SC * 2 // 16
LUT_BLK = 5 * LUT_PER_TROW

_MESH = plsc.VectorSubcoreMesh(
    core_axis_name="c", subcore_axis_name="s", num_cores=2, num_subcores=16)
_SC_PARAMS = pltpu.CompilerParams(needs_layout_passes=False)


def _axis_tables(n_pix, half, n_tiles):
    pos = np.arange(n_pix)
    m = pos // half
    last = 2 * n_tiles - 1
    interior = (m > 0) & (m < last)
    p = np.clip((m - 1) // 2, 0, n_tiles - 2)
    i0 = np.where(m == 0, 0, np.where(m == last, n_tiles - 1, p))
    i1 = np.where(m == 0, 0, np.where(m == last, n_tiles - 1,
                                      np.minimum(p + 1, n_tiles - 1)))
    r = (pos - (2 * p + 1) * half).astype(np.float32)
    denom = np.float32(2 * half - 1)
    w = np.where(interior, (denom - r) / denom, np.float32(1.0)).astype(np.float32)
    ci0 = i0.astype(np.int32) * NBINS
    ci1 = i1.astype(np.int32) * NBINS
    cpk = (ci0 | (ci1 << 16)).astype(np.int32)
    return cpk, w


def _clahe_body(x_hbm, cpk_h, wx_h, out_hbm,
                inb0, inb1, outb0, outb1, hist, lutb, lutv, pairb, cpkv, wxv,
                lut_sh, sem0, sem1, semw, sem_o0, sem_o1):
    cid = lax.axis_index("c")
    sid = lax.axis_index("s")
    ones = jnp.full((16,), 1.0, jnp.float32)
    iota_f = lax.iota(jnp.int32, 16).astype(jnp.float32)
    last_idx = jnp.full((16,), 15, jnp.int32)

    ltr0 = sid * TROW_PER_TEC
    gbase0 = (cid * IMG_PER_SC * GH + ltr0) * TROW_PIX
    pltpu.async_copy(x_hbm.at[pl.ds(gbase0, K_CHUNK)], inb0, sem0)
    pltpu.async_copy(x_hbm.at[pl.ds(gbase0 + K_CHUNK, K_CHUNK)], inb1, sem1)

    def per_tilerow(t, _):
        ltr = ltr0 + t
        base = (cid * IMG_PER_SC * GH + ltr) * TROW_PIX

        def zero(k, _c):
            for u in range(4):
                hist[pl.ds(k * 64 + u * 16, 16)] = jnp.zeros((16,), jnp.float32)
            return 0
        lax.fori_loop(0, LUT_PER_TROW // 64, zero, 0)

        def scat_chunk(inb):
            @plsc.parallel_loop(0, K_CHUNK // 64, unroll=4)
            def blk(bk):
                i = bk % 8
                ib = i * NBINS
                for u in range(4):
                    x = inb[pl.ds(bk * 64 + u * 16, 16)]
                    bins = jnp.clip((x * 256.0).astype(jnp.int32), 0, 255)
                    plsc.addupdate_scatter(hist, [bins + ib], ones)

        pltpu.make_async_copy(x_hbm.at[pl.ds(base, K_CHUNK)], inb0, sem0).wait()
        scat_chunk(inb0)
        pltpu.make_async_copy(
            x_hbm.at[pl.ds(base + K_CHUNK, K_CHUNK)], inb1, sem1).wait()
        scat_chunk(inb1)

        @pl.when(t < TROW_PER_TEC - 1)
        def _prefetch():
            nbase = base + TROW_PIX
            pltpu.async_copy(x_hbm.at[pl.ds(nbase, K_CHUNK)], inb0, sem0)
            pltpu.async_copy(
                x_hbm.at[pl.ds(nbase + K_CHUNK, K_CHUNK)], inb1, sem1)

        @pl.when(t > 0)
        def _drain_lut():
            pltpu.make_async_copy(
                lutb, lut_sh.at[pl.ds(ltr * LUT_PER_TROW, LUT_PER_TROW)],
                semw).wait()

        @plsc.parallel_loop(0, GW)
        def per_tile(i):
            hbase = i * NBINS

            def clip_sum(k, acc):
                h = jnp.minimum(hist[pl.ds(hbase + k * 16, 16)], MAXV)
                hist[pl.ds(hbase + k * 16, 16)] = h
                return acc + h
            accv = lax.fori_loop(0, 16, clip_sum,
                                 jnp.zeros((16,), jnp.float32))
            clipped = float(PIXELS) - jnp.sum(accv)
            q = (clipped * (1.0 / NBINS)).astype(jnp.int32).astype(jnp.float32)
            residual = clipped - q * float(NBINS)

            def cdf(k, carry):
                h = hist[pl.ds(hbase + k * 16, 16)]
                ind = jnp.where(iota_f + k.astype(jnp.float32) * 16.0 < residual,
                                1.0, 0.0)
                h2 = h + q + ind
                cs = plsc.cumsum(h2)
                csc = cs + carry
                lv = jnp.clip(csc * LUT_SCALE, 0.0, 255.0)
                lutb[pl.ds(hbase + k * 16, 16)] = (
                    lv.astype(jnp.int32).astype(jnp.float32))
                return lax.gather(
                    csc, last_idx[:, None],
                    lax.GatherDimensionNumbers(
                        offset_dims=(), collapsed_slice_dims=(0,),
                        start_index_map=(0,)),
                    (1,), mode=lax.GatherScatterMode.PROMISE_IN_BOUNDS)
            lax.fori_loop(0, 16, cdf, jnp.zeros((16,), jnp.float32))

        pltpu.async_copy(
            lutb, lut_sh.at[pl.ds(ltr * LUT_PER_TROW, LUT_PER_TROW)], semw)
        return 0
    lax.fori_loop(0, TROW_PER_TEC, per_tilerow, 0)
    last_ltr = ltr0 + TROW_PER_TEC - 1
    pltpu.make_async_copy(
        lutb, lut_sh.at[pl.ds(last_ltr * LUT_PER_TROW, LUT_PER_TROW)],
        semw).wait()

    plsc.subcore_barrier()

    pltpu.sync_copy(cpk_h, cpkv)
    pltpu.sync_copy(wx_h, wxv)
    ins = (inb0, inb1)
    outs = (outb0, outb1)
    sis = (sem0, sem1)
    sos = (sem_o0, sem_o1)

    def per_half(hq, _):
        hw = sid * HALF_PER_TEC + hq
        img_l = hw // 2
        half = hw % 2
        jbase = half * 3
        pixbase = (cid * IMG_PER_SC + img_l) * IMG_PIX + half * (IMG_PIX // 2)
        d_in = [None, None]
        d_out = [None, None]
        d_in[0] = pltpu.async_copy(
            x_hbm.at[pl.ds(pixbase, K_CHUNK)], ins[0], sis[0])
        d_lut = pltpu.async_copy(
            lut_sh.at[pl.ds((img_l * GH + jbase) * LUT_PER_TROW, LUT_BLK)],
            lutv, semw)

        for ch in range(NCHUNK_HALF):
            bi = ch % 2
            if ch + 1 < NCHUNK_HALF:
                d_in[1 - bi] = pltpu.async_copy(
                    x_hbm.at[pl.ds(pixbase + (ch + 1) * K_CHUNK, K_CHUNK)],
                    ins[1 - bi], sis[1 - bi])
            m = half * 8 + ch
            p = jnp.clip((m - 1) // 2, 0, GH - 2)
            j0 = jnp.where(m == 0, 0, jnp.where(m == 15, GH - 1, p))
            j1 = jnp.where(m == 0, 0,
                           jnp.where(m == 15, GH - 1,
                                     jnp.minimum(p + 1, GH - 1)))
            interior = jnp.logical_and(m > 0, m < 15)
            ro0 = (j0 - jbase) * LUT_PER_TROW
            ro1 = (j1 - jbase) * LUT_PER_TROW
            rr0 = (half * 256 + ch * 32 - (2 * p + 1) * HALF).astype(jnp.float32)

            if ch == 0:
                d_lut.wait()

            @plsc.parallel_loop(0, LUT_PER_TROW // 64, unroll=2)
            def mkpair(k):
                for u in range(4):
                    o = k * 64 + u * 16
                    a = lutv[pl.ds(ro0 + o, 16)]
                    b = lutv[pl.ds(ro1 + o, 16)]
                    w = plsc.bitcast(
                        plsc.pack(a, b, format=plsc.PackFormat.INTERLEAVED),
                        jnp.int32)
                    pairb[pl.ds(o, 16)] = w

            if d_out[bi] is not None:
                d_out[bi].wait()
            d_in[bi].wait()
            outb = outs[bi]
            inb = ins[bi]

            def per_row(rl, _c):
                rlf = rl.astype(jnp.float32)
                wy = jnp.where(interior, (63.0 - (rr0 + rlf)) * (1.0 / 63.0),
                               1.0)
                wy1 = 1.0 - wy
                rowb = rl * W

                @plsc.parallel_loop(0, W // 16, unroll=4)
                def pg(g):
                    x = inb[pl.ds(rowb + g * 16, 16)]
                    v = (x * 255.0).astype(jnp.int32)
                    cpk = cpkv[pl.ds(g * 16, 16)]
                    c0 = jnp.bitwise_and(cpk, 0xFFFF)
                    c1 = lax.shift_right_logical(cpk, 16)
                    pw0 = plsc.load_gather(pairb, [v + c0])
                    pw1 = plsc.load_gather(pairb, [v + c1])
                    o00, o10 = plsc.unpack(
                        plsc.bitcast(pw0, jnp.bfloat16),
                        format=plsc.PackFormat.INTERLEAVED)
                    o01, o11 = plsc.unpack(
                        plsc.bitcast(pw1, jnp.bfloat16),
                        format=plsc.PackFormat.INTERLEAVED)
                    wx = wxv[pl.ds(g * 16, 16)]
                    wx1 = 1.0 - wx
                    m0 = wy * o00 + wy1 * o10
                    m1 = wy * o01 + wy1 * o11
                    outb[pl.ds(rowb + g * 16, 16)] = (
                        (wx * m0 + wx1 * m1) * (1.0 / 255.0))
                return 0
            lax.fori_loop(0, K_CHUNK // W, per_row, 0)

            d_out[bi] = pltpu.async_copy(
                outb, out_hbm.at[pl.ds(pixbase + ch * K_CHUNK, K_CHUNK)],
                sos[bi])
        d_out[0].wait()
        d_out[1].wait()
        return 0
    lax.fori_loop(0, HALF_PER_TEC, per_half, 0)


_clahe = pl.kernel(
    _clahe_body,
    out_type=jax.ShapeDtypeStruct((NIMG * IMG_PIX,), jnp.float32),
    mesh=_MESH,
    compiler_params=_SC_PARAMS,
    scratch_types=[
        pltpu.VMEM((K_CHUNK,), jnp.float32),
        pltpu.VMEM((K_CHUNK,), jnp.float32),
        pltpu.VMEM((K_CHUNK,), jnp.float32),
        pltpu.VMEM((K_CHUNK,), jnp.float32),
        pltpu.VMEM((LUT_PER_TROW,), jnp.float32),
        pltpu.VMEM((LUT_PER_TROW,), jnp.float32),
        pltpu.VMEM((LUT_BLK,), jnp.float32),
        pltpu.VMEM((LUT_PER_TROW,), jnp.int32),
        pltpu.VMEM((W,), jnp.int32),
        pltpu.VMEM((W,), jnp.float32),
        pltpu.VMEM_SHARED((LUT_SC,), jnp.float32),
        pltpu.SemaphoreType.DMA,
        pltpu.SemaphoreType.DMA,
        pltpu.SemaphoreType.DMA,
        pltpu.SemaphoreType.DMA,
        pltpu.SemaphoreType.DMA,
    ],
)


def kernel(input):
    x_flat = input.reshape(-1)
    cpk, wx = _axis_tables(W, HALF, GW)
    out = _clahe(x_flat, jnp.asarray(cpk), jnp.asarray(wx))
    return out.reshape(input.shape)


# speedup vs baseline: 2226.3759x; 1.0014x over previous
import jax
import jax.numpy as jnp
import numpy as np
from jax import lax
from jax.experimental import pallas as pl
from jax.experimental.pallas import tpu as pltpu
from jax.experimental.pallas import tpu_sc as plsc

B, C, H, W = 16, 3, 512, 512
GH = GW = 8
TS = 64
HALF = TS // 2
NBINS = 256
PIXELS = TS * TS
MAXV = 640.0
LUT_SCALE = (NBINS - 1) / PIXELS

NIMG = B * C
IMG_PER_SC = NIMG // 2
IMG_PIX = H * W
TROW_PIX = TS * W
TROW_PER_TEC = IMG_PER_SC * GH // 16
LUT_PER_TROW = GW * NBINS
LUT_SC = IMG_PER_SC * GH * LUT_PER_TROW

K_CHUNK = 32 * W
NCHUNK_HALF = IMG_PIX // 2 // K_CHUNK
HALF_PER_TEC = IMG_PER_SC * 2 // 16
LUT_BLK = 5 * LUT_PER_TROW

_MESH = plsc.VectorSubcoreMesh(
    core_axis_name="c", subcore_axis_name="s", num_cores=2, num_subcores=16)
_SC_PARAMS = pltpu.CompilerParams(needs_layout_passes=False)


def _axis_tables(n_pix, half, n_tiles):
    pos = np.arange(n_pix)
    m = pos // half
    last = 2 * n_tiles - 1
    interior = (m > 0) & (m < last)
    p = np.clip((m - 1) // 2, 0, n_tiles - 2)
    i0 = np.where(m == 0, 0, np.where(m == last, n_tiles - 1, p))
    i1 = np.where(m == 0, 0, np.where(m == last, n_tiles - 1,
                                      np.minimum(p + 1, n_tiles - 1)))
    r = (pos - (2 * p + 1) * half).astype(np.float32)
    denom = np.float32(2 * half - 1)
    w = np.where(interior, (denom - r) / denom, np.float32(1.0)).astype(np.float32)
    ci0 = i0.astype(np.int32) * NBINS
    ci1 = i1.astype(np.int32) * NBINS
    cpk = (ci0 | (ci1 << 16)).astype(np.int32)
    return cpk, w


def _clahe_body(x_hbm, cpk_h, wx_h, out_hbm,
                inb0, inb1, outb0, outb1, hist, lutb, lutv, pairb, cpkv, wxv,
                lut_sh, sem0, sem1, semw, sem_o0, sem_o1):
    cid = lax.axis_index("c")
    sid = lax.axis_index("s")
    ones = jnp.full((16,), 1.0, jnp.float32)
    iota_f = lax.iota(jnp.int32, 16).astype(jnp.float32)
    last_idx = jnp.full((16,), 15, jnp.int32)

    ltr0 = sid * TROW_PER_TEC
    gbase0 = (cid * IMG_PER_SC * GH + ltr0) * TROW_PIX
    pltpu.async_copy(x_hbm.at[pl.ds(gbase0, K_CHUNK)], inb0, sem0)
    pltpu.async_copy(x_hbm.at[pl.ds(gbase0 + K_CHUNK, K_CHUNK)], inb1, sem1)

    def per_tilerow(t, _):
        ltr = ltr0 + t
        base = (cid * IMG_PER_SC * GH + ltr) * TROW_PIX

        def zero(k, _c):
            for u in range(4):
                hist[pl.ds(k * 64 + u * 16, 16)] = jnp.zeros((16,), jnp.float32)
            return 0
        lax.fori_loop(0, LUT_PER_TROW // 64, zero, 0)

        def scat_chunk(inb):
            @plsc.parallel_loop(0, K_CHUNK // 64, unroll=4)
            def blk(bk):
                i = bk % 8
                ib = i * NBINS
                for u in range(4):
                    x = inb[pl.ds(bk * 64 + u * 16, 16)]
                    bins = jnp.clip((x * 256.0).astype(jnp.int32), 0, 255)
                    plsc.addupdate_scatter(hist, [bins + ib], ones)

        pltpu.make_async_copy(x_hbm.at[pl.ds(base, K_CHUNK)], inb0, sem0).wait()
        scat_chunk(inb0)
        pltpu.make_async_copy(
            x_hbm.at[pl.ds(base + K_CHUNK, K_CHUNK)], inb1, sem1).wait()
        scat_chunk(inb1)

        @pl.when(t < TROW_PER_TEC - 1)
        def _prefetch():
            nbase = base + TROW_PIX
            pltpu.async_copy(x_hbm.at[pl.ds(nbase, K_CHUNK)], inb0, sem0)
            pltpu.async_copy(
                x_hbm.at[pl.ds(nbase + K_CHUNK, K_CHUNK)], inb1, sem1)

        @pl.when(t > 0)
        def _drain_lut():
            pltpu.make_async_copy(
                lutb, lut_sh.at[pl.ds(ltr * LUT_PER_TROW, LUT_PER_TROW)],
                semw).wait()

        @plsc.parallel_loop(0, GW)
        def per_tile(i):
            hbase = i * NBINS

            def clip_sum(k, acc):
                h = jnp.minimum(hist[pl.ds(hbase + k * 16, 16)], MAXV)
                hist[pl.ds(hbase + k * 16, 16)] = h
                return acc + h
            accv = lax.fori_loop(0, 16, clip_sum,
                                 jnp.zeros((16,), jnp.float32))
            clipped = float(PIXELS) - jnp.sum(accv)
            q = (clipped * (1.0 / NBINS)).astype(jnp.int32).astype(jnp.float32)
            residual = clipped - q * float(NBINS)

            def cdf(k, carry):
                h = hist[pl.ds(hbase + k * 16, 16)]
                ind = jnp.where(iota_f + k.astype(jnp.float32) * 16.0 < residual,
                                1.0, 0.0)
                h2 = h + q + ind
                cs = plsc.cumsum(h2)
                csc = cs + carry
                lv = jnp.clip(csc * LUT_SCALE, 0.0, 255.0)
                lutb[pl.ds(hbase + k * 16, 16)] = (
                    lv.astype(jnp.int32).astype(jnp.float32))
                return lax.gather(
                    csc, last_idx[:, None],
                    lax.GatherDimensionNumbers(
                        offset_dims=(), collapsed_slice_dims=(0,),
                        start_index_map=(0,)),
                    (1,), mode=lax.GatherScatterMode.PROMISE_IN_BOUNDS)
            lax.fori_loop(0, 16, cdf, jnp.zeros((16,), jnp.float32))

        pltpu.async_copy(
            lutb, lut_sh.at[pl.ds(ltr * LUT_PER_TROW, LUT_PER_TROW)], semw)
        return 0
    lax.fori_loop(0, TROW_PER_TEC, per_tilerow, 0)
    last_ltr = ltr0 + TROW_PER_TEC - 1
    pltpu.make_async_copy(
        lutb, lut_sh.at[pl.ds(last_ltr * LUT_PER_TROW, LUT_PER_TROW)],
        semw).wait()

    plsc.subcore_barrier()

    pltpu.sync_copy(cpk_h, cpkv)
    pltpu.sync_copy(wx_h, wxv)
    ins = (inb0, inb1)
    outs = (outb0, outb1)
    sis = (sem0, sem1)
    sos = (sem_o0, sem_o1)

    def per_half(hq, _):
        hw = sid * HALF_PER_TEC + hq
        img_l = hw // 2
        half = hw % 2
        jbase = half * 3
        pixbase = (cid * IMG_PER_SC + img_l) * IMG_PIX + half * (IMG_PIX // 2)
        d_in = [None, None]
        d_out = [None, None]
        d_in[0] = pltpu.async_copy(
            x_hbm.at[pl.ds(pixbase, K_CHUNK)], ins[0], sis[0])
        d_lut = pltpu.async_copy(
            lut_sh.at[pl.ds((img_l * GH + jbase) * LUT_PER_TROW, LUT_BLK)],
            lutv, semw)

        for ch in range(NCHUNK_HALF):
            bi = ch % 2
            if ch + 1 < NCHUNK_HALF:
                d_in[1 - bi] = pltpu.async_copy(
                    x_hbm.at[pl.ds(pixbase + (ch + 1) * K_CHUNK, K_CHUNK)],
                    ins[1 - bi], sis[1 - bi])
            m = half * 8 + ch
            p = jnp.clip((m - 1) // 2, 0, GH - 2)
            j0 = jnp.where(m == 0, 0, jnp.where(m == 15, GH - 1, p))
            j1 = jnp.where(m == 0, 0,
                           jnp.where(m == 15, GH - 1,
                                     jnp.minimum(p + 1, GH - 1)))
            interior = jnp.logical_and(m > 0, m < 15)
            ro0 = (j0 - jbase) * LUT_PER_TROW
            ro1 = (j1 - jbase) * LUT_PER_TROW
            rr0 = (half * 256 + ch * 32 - (2 * p + 1) * HALF).astype(jnp.float32)

            if ch == 0:
                d_lut.wait()

            @plsc.parallel_loop(0, LUT_PER_TROW // 64, unroll=2)
            def mkpair(k):
                for u in range(4):
                    o = k * 64 + u * 16
                    a = lutv[pl.ds(ro0 + o, 16)]
                    b = lutv[pl.ds(ro1 + o, 16)]
                    w = plsc.bitcast(
                        plsc.pack(a, b, format=plsc.PackFormat.INTERLEAVED),
                        jnp.int32)
                    pairb[pl.ds(o, 16)] = w

            if d_out[bi] is not None:
                d_out[bi].wait()
            d_in[bi].wait()
            outb = outs[bi]
            inb = ins[bi]

            @plsc.parallel_loop(0, K_CHUNK // W)
            def per_row(rl):
                rlf = rl.astype(jnp.float32)
                wy = jnp.where(interior, (63.0 - (rr0 + rlf)) * (1.0 / 63.0),
                               1.0)
                wy1 = 1.0 - wy
                rowb = rl * W

                @plsc.parallel_loop(0, W // 16, unroll=4)
                def pg(g):
                    x = inb[pl.ds(rowb + g * 16, 16)]
                    v = (x * 255.0).astype(jnp.int32)
                    cpk = cpkv[pl.ds(g * 16, 16)]
                    c0 = jnp.bitwise_and(cpk, 0xFFFF)
                    c1 = lax.shift_right_logical(cpk, 16)
                    pw0 = plsc.load_gather(pairb, [v + c0])
                    pw1 = plsc.load_gather(pairb, [v + c1])
                    o00, o10 = plsc.unpack(
                        plsc.bitcast(pw0, jnp.bfloat16),
                        format=plsc.PackFormat.INTERLEAVED)
                    o01, o11 = plsc.unpack(
                        plsc.bitcast(pw1, jnp.bfloat16),
                        format=plsc.PackFormat.INTERLEAVED)
                    wx = wxv[pl.ds(g * 16, 16)]
                    wx1 = 1.0 - wx
                    m0 = wy * o00 + wy1 * o10
                    m1 = wy * o01 + wy1 * o11
                    outb[pl.ds(rowb + g * 16, 16)] = (
                        (wx * m0 + wx1 * m1) * (1.0 / 255.0))

            d_out[bi] = pltpu.async_copy(
                outb, out_hbm.at[pl.ds(pixbase + ch * K_CHUNK, K_CHUNK)],
                sos[bi])
        d_out[0].wait()
        d_out[1].wait()
        return 0
    lax.fori_loop(0, HALF_PER_TEC, per_half, 0)


_clahe = pl.kernel(
    _clahe_body,
    out_type=jax.ShapeDtypeStruct((NIMG * IMG_PIX,), jnp.float32),
    mesh=_MESH,
    compiler_params=_SC_PARAMS,
    scratch_types=[
        pltpu.VMEM((K_CHUNK,), jnp.float32),
        pltpu.VMEM((K_CHUNK,), jnp.float32),
        pltpu.VMEM((K_CHUNK,), jnp.float32),
        pltpu.VMEM((K_CHUNK,), jnp.float32),
        pltpu.VMEM((LUT_PER_TROW,), jnp.float32),
        pltpu.VMEM((LUT_PER_TROW,), jnp.float32),
        pltpu.VMEM((LUT_BLK,), jnp.float32),
        pltpu.VMEM((LUT_PER_TROW,), jnp.int32),
        pltpu.VMEM((W,), jnp.int32),
        pltpu.VMEM((W,), jnp.float32),
        pltpu.VMEM_SHARED((LUT_SC,), jnp.float32),
        pltpu.SemaphoreType.DMA,
        pltpu.SemaphoreType.DMA,
        pltpu.SemaphoreType.DMA,
        pltpu.SemaphoreType.DMA,
        pltpu.SemaphoreType.DMA,
    ],
)


def kernel(input):
    x_flat = input.reshape(-1)
    cpk, wx = _axis_tables(W, HALF, GW)
    out = _clahe(x_flat, jnp.asarray(cpk), jnp.asarray(wx))
    return out.reshape(input.shape)
